# Initial kernel scaffold; baseline (speedup 1.0000x reference)
import jax
import jax.numpy as jnp
from jax.experimental import pallas as pl


def kernel(input_ids, row_pos_from, row_pos_to, col_pos_from, col_pos_to, row_embedding, col_embedding):
    raise NotImplementedError("write your pallas kernel here")


# speedup vs baseline: 1.1201x; 1.1201x over previous
import functools

import jax
import jax.numpy as jnp
from jax import lax
from jax.experimental import pallas as pl
from jax.experimental.pallas import tpu as pltpu
from jax.experimental.pallas import tpu_sc as plsc

EMBED = 768
DEPTH = 128
LANES = 16

_NW = 32
_CK = 32


def _rne_to_int(x):
    s = x + 0.5
    t = s.astype(jnp.int32)
    d = s - t.astype(jnp.float32)
    nonint = jnp.minimum((d * 16777216.0).astype(jnp.int32), 1)
    return t - ((1 - nonint) & t & 1)


def _mean_idx(f, t):
    s = f + t
    h = s >> 1
    i = h + ((s & h) & 1)
    return jnp.minimum(jnp.maximum(i, 0), DEPTH - 1)


def _body(tpw, in_hbm, rpf_hbm, rpt_hbm, cpf_hbm, cpt_hbm, rtab_hbm,
          ctab_hbm, out_hbm, posb, ridx, cidx, rowb, colb, inb,
          sem0, sem1, sem2):
    wid = lax.axis_index("s") * 2 + lax.axis_index("c")
    base = wid * tpw

    def chunk(c, carry):
        t0 = base + c * _CK
        pltpu.sync_copy(rpf_hbm.at[pl.ds(t0, _CK)], posb.at[0])
        pltpu.sync_copy(rpt_hbm.at[pl.ds(t0, _CK)], posb.at[1])
        pltpu.sync_copy(cpf_hbm.at[pl.ds(t0, _CK)], posb.at[2])
        pltpu.sync_copy(cpt_hbm.at[pl.ds(t0, _CK)], posb.at[3])
        for g in range(_CK // LANES):
            sl = pl.ds(g * LANES, LANES)
            rf = _rne_to_int(posb[0, sl] * float(DEPTH))
            rt = _rne_to_int(posb[1, sl] * float(DEPTH))
            cf = _rne_to_int(posb[2, sl] * float(DEPTH))
            ct = _rne_to_int(posb[3, sl] * float(DEPTH))
            ridx[sl] = _mean_idx(rf, rt)
            cidx[sl] = _mean_idx(cf, ct)
        cp_r = pltpu.async_copy(rtab_hbm.at[ridx], rowb, sem0)
        cp_c = pltpu.async_copy(ctab_hbm.at[cidx], colb, sem1)
        cp_i = pltpu.async_copy(in_hbm.at[pl.ds(t0, _CK)], inb, sem2)
        cp_r.wait()
        cp_c.wait()
        cp_i.wait()

        def tok(t, carry2):
            for d in range(EMBED // LANES):
                sl = pl.ds(d * LANES, LANES)
                inb[t, sl] = inb[t, sl] + rowb[t, sl] + colb[t, sl]
            return carry2

        lax.fori_loop(0, _CK, tok, 0)
        pltpu.sync_copy(inb, out_hbm.at[pl.ds(t0, _CK)])
        return carry

    lax.fori_loop(0, tpw // _CK, chunk, 0)


def kernel(input_ids, row_pos_from, row_pos_to, col_pos_from, col_pos_to,
           row_embedding, col_embedding):
    b, n, e = input_ids.shape
    t = b * n
    assert e == EMBED and t % (_NW * _CK) == 0
    tpw = t // _NW

    x = input_ids.reshape(t, e)
    rpf = row_pos_from.reshape(t)
    rpt = row_pos_to.reshape(t)
    cpf = col_pos_from.reshape(t)
    cpt = col_pos_to.reshape(t)

    mesh = plsc.VectorSubcoreMesh(core_axis_name="c", subcore_axis_name="s")
    run = functools.partial(
        pl.kernel,
        mesh=mesh,
        out_type=jax.ShapeDtypeStruct((t, e), jnp.float32),
        scratch_types=[
            pltpu.VMEM((4, _CK), jnp.float32),
            pltpu.VMEM((_CK,), jnp.int32),
            pltpu.VMEM((_CK,), jnp.int32),
            pltpu.VMEM((_CK, EMBED), jnp.float32),
            pltpu.VMEM((_CK, EMBED), jnp.float32),
            pltpu.VMEM((_CK, EMBED), jnp.float32),
            pltpu.SemaphoreType.DMA,
            pltpu.SemaphoreType.DMA,
            pltpu.SemaphoreType.DMA,
        ],
    )(functools.partial(_body, tpw))
    out = run(x, rpf, rpt, cpf, cpt, row_embedding, col_embedding)
    return out.reshape(b, n, e)


# speedup vs baseline: 1.8009x; 1.6077x over previous
import functools

import jax
import jax.numpy as jnp
from jax import lax
from jax.experimental import pallas as pl
from jax.experimental.pallas import tpu as pltpu
from jax.experimental.pallas import tpu_sc as plsc

EMBED = 768
DEPTH = 128
LANES = 16

_NW = 32
_CK = 8
_NS = 4


def _rne_to_int(x):
    s = x + 0.5
    t = s.astype(jnp.int32)
    d = s - t.astype(jnp.float32)
    nonint = jnp.minimum((d * 16777216.0).astype(jnp.int32), 1)
    return t - ((1 - nonint) & t & 1)


def _mean_idx(f, t):
    s = f + t
    h = s >> 1
    i = h + ((s & h) & 1)
    return jnp.minimum(jnp.maximum(i, 0), DEPTH - 1)


def _body(tpw, in_hbm, rpf_hbm, rpt_hbm, cpf_hbm, cpt_hbm, rtab_hbm,
          ctab_hbm, out_hbm, posb, ridx, cidx, *slotrefs):
    rowb = slotrefs[0:_NS]
    colb = slotrefs[_NS:2 * _NS]
    inb = slotrefs[2 * _NS:3 * _NS]
    semg = slotrefs[3 * _NS:4 * _NS]
    semo = slotrefs[4 * _NS:5 * _NS]

    wid = lax.axis_index("s") * 2 + lax.axis_index("c")
    base = wid * tpw
    nc = tpw // _CK

    pltpu.sync_copy(rpf_hbm.at[pl.ds(base, tpw)], posb.at[0])
    pltpu.sync_copy(rpt_hbm.at[pl.ds(base, tpw)], posb.at[1])
    pltpu.sync_copy(cpf_hbm.at[pl.ds(base, tpw)], posb.at[2])
    pltpu.sync_copy(cpt_hbm.at[pl.ds(base, tpw)], posb.at[3])

    def idx_body(g, carry):
        sl = pl.ds(g * LANES, LANES)
        rf = _rne_to_int(posb[0, sl] * float(DEPTH))
        rt = _rne_to_int(posb[1, sl] * float(DEPTH))
        cf = _rne_to_int(posb[2, sl] * float(DEPTH))
        ct = _rne_to_int(posb[3, sl] * float(DEPTH))
        ridx[sl] = _mean_idx(rf, rt)
        cidx[sl] = _mean_idx(cf, ct)
        return carry

    lax.fori_loop(0, tpw // LANES, idx_body, 0)

    def issue(cc, s):
        t0 = base + cc * _CK
        pltpu.async_copy(rtab_hbm.at[ridx.at[pl.ds(cc * _CK, _CK)]],
                         rowb[s], semg[s])
        pltpu.async_copy(ctab_hbm.at[cidx.at[pl.ds(cc * _CK, _CK)]],
                         colb[s], semg[s])
        pltpu.async_copy(in_hbm.at[pl.ds(t0, _CK)], inb[s], semg[s])

    def drain_out(s):
        pltpu.make_async_copy(rowb[s], out_hbm.at[pl.ds(base, _CK)],
                              semo[s]).wait()

    def compute(cc, s):
        t0 = base + cc * _CK
        src = in_hbm.at[pl.ds(t0, _CK)]
        pltpu.make_async_copy(src, rowb[s], semg[s]).wait()
        pltpu.make_async_copy(src, colb[s], semg[s]).wait()
        pltpu.make_async_copy(src, inb[s], semg[s]).wait()

        def tok(t, carry):
            for d in range(EMBED // LANES):
                sl = pl.ds(d * LANES, LANES)
                plsc.addupdate(rowb[s].at[t, sl], inb[s][t, sl] + colb[s][t, sl])
            return carry

        lax.fori_loop(0, _CK, tok, 0)
        pltpu.async_copy(rowb[s], out_hbm.at[pl.ds(t0, _CK)], semo[s])

    issue(0, 0)
    issue(1, 1)

    def pipe(c4, carry):
        for s in range(_NS):
            c = c4 * _NS + s
            cn = c + 2
            sn = (s + 2) % _NS

            @pl.when(cn < nc)
            def _issue_ahead():
                @pl.when(cn >= _NS)
                def _drain_prev():
                    drain_out(sn)

                issue(cn, sn)

            compute(c, s)
        return carry

    lax.fori_loop(0, nc // _NS, pipe, 0)
    for s in range(_NS):
        drain_out(s)


def kernel(input_ids, row_pos_from, row_pos_to, col_pos_from, col_pos_to,
           row_embedding, col_embedding):
    b, n, e = input_ids.shape
    t = b * n
    assert e == EMBED and t % (_NW * _NS * _CK) == 0
    tpw = t // _NW

    x = input_ids.reshape(t, e)
    rpf = row_pos_from.reshape(t)
    rpt = row_pos_to.reshape(t)
    cpf = col_pos_from.reshape(t)
    cpt = col_pos_to.reshape(t)

    slot_types = (
        [pltpu.VMEM((_CK, EMBED), jnp.float32) for _ in range(3 * _NS)]
        + [pltpu.SemaphoreType.DMA for _ in range(2 * _NS)]
    )
    mesh = plsc.VectorSubcoreMesh(core_axis_name="c", subcore_axis_name="s")
    run = functools.partial(
        pl.kernel,
        mesh=mesh,
        out_type=jax.ShapeDtypeStruct((t, e), jnp.float32),
        scratch_types=[
            pltpu.VMEM((4, tpw), jnp.float32),
            pltpu.VMEM((tpw,), jnp.int32),
            pltpu.VMEM((tpw,), jnp.int32),
        ] + slot_types,
    )(functools.partial(_body, tpw))
    out = run(x, rpf, rpt, cpf, cpt, row_embedding, col_embedding)
    return out.reshape(b, n, e)
